# baseline (speedup 1.0000x reference)
import dataclasses
import functools

import jax
import jax.numpy as jnp
from jax import lax
from jax.experimental import pallas as pl
from jax.experimental.pallas import tpu as pltpu
from jax.experimental.pallas import tpu_sc as plsc

NC = 2
NS = 16
NW = NC * NS
LANES = 16

SEQ_LEN = 100000
N = 16384 * 128
PER_W = N // NW
CHUNK = 4096
NCHUNK = PER_W // CHUNK
NBUF = 2


def _sc_kernel(idx_hbm, src_hbm, seq_hbm, out_hbm,
               seq_sh, seq_v,
               idx_v0, idx_v1, src_v0, src_v1, out_v0, out_v1,
               in_sems, out_sems, stage_sem):
    cid = lax.axis_index("c")
    sid = lax.axis_index("s")
    wid = sid * NC + cid

    idx_bufs = (idx_v0, idx_v1)
    src_bufs = (src_v0, src_v1)
    out_bufs = (out_v0, out_v1)

    @pl.when(sid == 0)
    def _stage():
        pltpu.async_copy(seq_hbm, seq_sh, stage_sem).wait()

    plsc.subcore_barrier()
    pltpu.async_copy(seq_sh, seq_v, stage_sem).wait()

    base0 = wid * PER_W

    def start_in(ci, b):
        base = base0 + ci * CHUNK
        pltpu.make_async_copy(
            idx_hbm.at[pl.ds(base, CHUNK)], idx_bufs[b], in_sems.at[b]
        ).start()
        pltpu.make_async_copy(
            src_hbm.at[pl.ds(base, CHUNK)], src_bufs[b], in_sems.at[b]
        ).start()

    def wait_in(b):
        pltpu.make_async_copy(
            idx_hbm.at[pl.ds(0, CHUNK)], idx_bufs[b], in_sems.at[b]
        ).wait()
        pltpu.make_async_copy(
            src_hbm.at[pl.ds(0, CHUNK)], src_bufs[b], in_sems.at[b]
        ).wait()

    def start_out(ci, b):
        base = base0 + ci * CHUNK
        pltpu.make_async_copy(
            out_bufs[b], out_hbm.at[pl.ds(base, CHUNK)], out_sems.at[b]
        ).start()

    def wait_out(b):
        pltpu.make_async_copy(
            out_bufs[b], out_hbm.at[pl.ds(0, CHUNK)], out_sems.at[b]
        ).wait()

    def compute(b):
        idx_v, src_v, out_v = idx_bufs[b], src_bufs[b], out_bufs[b]

        @pl.loop(0, CHUNK, step=4 * LANES)
        def _vec(o):
            for u in range(4):
                i16 = idx_v[pl.ds(o + u * LANES, LANES)]
                s16 = src_v[pl.ds(o + u * LANES, LANES)]
                v16 = plsc.load_gather(seq_v, [i16])
                out_v[pl.ds(o + u * LANES, LANES)] = jnp.where(s16 > v16, 1.0, 0.0)

    start_in(0, 0)
    start_in(1, 1)

    @pl.loop(0, NCHUNK, step=NBUF)
    def _chunks(ci):
        for b in range(NBUF):
            wait_in(b)

            @pl.when(ci + b >= NBUF)
            def _():
                wait_out(b)

            compute(b)
            start_out(ci + b, b)

            @pl.when(ci + b + NBUF < NCHUNK)
            def _():
                start_in(ci + b + NBUF, b)

    for b in range(NBUF):
        wait_out(b)


@jax.jit
def kernel(rng_idx, source, rng_seq):
    shape = source.shape
    idx = rng_idx.astype(jnp.int32).reshape(-1)
    src = source.reshape(-1)

    cp = pltpu.CompilerParams()
    if "needs_layout_passes" in pltpu.CompilerParams.__dataclass_fields__:
        cp = dataclasses.replace(cp, needs_layout_passes=False)

    mesh = plsc.VectorSubcoreMesh(core_axis_name="c", subcore_axis_name="s")
    run = pl.kernel(
        _sc_kernel,
        out_type=jax.ShapeDtypeStruct((N,), jnp.float32),
        mesh=mesh,
        scratch_types=[
            pltpu.VMEM_SHARED((SEQ_LEN,), jnp.float32),
            pltpu.VMEM((SEQ_LEN,), jnp.float32),
            pltpu.VMEM((CHUNK,), jnp.int32),
            pltpu.VMEM((CHUNK,), jnp.int32),
            pltpu.VMEM((CHUNK,), jnp.float32),
            pltpu.VMEM((CHUNK,), jnp.float32),
            pltpu.VMEM((CHUNK,), jnp.float32),
            pltpu.VMEM((CHUNK,), jnp.float32),
            pltpu.SemaphoreType.DMA((NBUF,)),
            pltpu.SemaphoreType.DMA((NBUF,)),
            pltpu.SemaphoreType.DMA,
        ],
        compiler_params=cp,
    )
    out = run(idx, src, rng_seq)
    return out.reshape(shape)


# speedup vs baseline: 1.2762x; 1.2762x over previous
import dataclasses
import functools

import jax
import jax.numpy as jnp
from jax import lax
from jax.experimental import pallas as pl
from jax.experimental.pallas import tpu as pltpu
from jax.experimental.pallas import tpu_sc as plsc

NC = 2
NS = 16
NW = NC * NS
LANES = 16

SEQ_LEN = 100000
N = 16384 * 128
PER_W = N // NW
CHUNK = 4096
NCHUNK = PER_W // CHUNK
NBUF = 2


def _sc_kernel(idx_hbm, src_hbm, seq_hbm, out_hbm,
               seq_sh, seq_v,
               idx_v0, idx_v1, src_v0, src_v1, out_v0, out_v1,
               in_sems, out_sems, stage_sem):
    cid = lax.axis_index("c")
    sid = lax.axis_index("s")
    wid = sid * NC + cid

    idx_bufs = (idx_v0, idx_v1)
    src_bufs = (src_v0, src_v1)
    out_bufs = (out_v0, out_v1)

    @pl.when(sid == 0)
    def _stage():
        pltpu.async_copy(seq_hbm, seq_sh, stage_sem).wait()

    plsc.subcore_barrier()
    pltpu.async_copy(seq_sh, seq_v, stage_sem).wait()

    base0 = wid * PER_W

    def start_in(ci, b):
        base = base0 + ci * CHUNK
        pltpu.make_async_copy(
            idx_hbm.at[pl.ds(base, CHUNK)], idx_bufs[b], in_sems.at[b]
        ).start()
        pltpu.make_async_copy(
            src_hbm.at[pl.ds(base, CHUNK)], src_bufs[b], in_sems.at[b]
        ).start()

    def wait_in(b):
        pltpu.make_async_copy(
            idx_hbm.at[pl.ds(0, CHUNK)], idx_bufs[b], in_sems.at[b]
        ).wait()
        pltpu.make_async_copy(
            src_hbm.at[pl.ds(0, CHUNK)], src_bufs[b], in_sems.at[b]
        ).wait()

    def start_out(ci, b):
        base = base0 + ci * CHUNK
        pltpu.make_async_copy(
            out_bufs[b], out_hbm.at[pl.ds(base, CHUNK)], out_sems.at[b]
        ).start()

    def wait_out(b):
        pltpu.make_async_copy(
            out_bufs[b], out_hbm.at[pl.ds(0, CHUNK)], out_sems.at[b]
        ).wait()

    def compute(b):
        idx_v, src_v, out_v = idx_bufs[b], src_bufs[b], out_bufs[b]

        @plsc.parallel_loop(0, CHUNK, step=LANES, unroll=8)
        def _vec(o):
            i16 = idx_v[pl.ds(o, LANES)]
            s16 = src_v[pl.ds(o, LANES)]
            v16 = plsc.load_gather(seq_v, [i16])
            out_v[pl.ds(o, LANES)] = jnp.where(s16 > v16, 1.0, 0.0)

    start_in(0, 0)
    start_in(1, 1)

    @pl.loop(0, NCHUNK, step=NBUF)
    def _chunks(ci):
        for b in range(NBUF):
            wait_in(b)

            @pl.when(ci + b >= NBUF)
            def _():
                wait_out(b)

            compute(b)
            start_out(ci + b, b)

            @pl.when(ci + b + NBUF < NCHUNK)
            def _():
                start_in(ci + b + NBUF, b)

    for b in range(NBUF):
        wait_out(b)


@jax.jit
def kernel(rng_idx, source, rng_seq):
    shape = source.shape
    idx = rng_idx.astype(jnp.int32).reshape(-1)
    src = source.reshape(-1)

    cp = pltpu.CompilerParams()
    if "needs_layout_passes" in pltpu.CompilerParams.__dataclass_fields__:
        cp = dataclasses.replace(cp, needs_layout_passes=False)

    mesh = plsc.VectorSubcoreMesh(core_axis_name="c", subcore_axis_name="s")
    run = pl.kernel(
        _sc_kernel,
        out_type=jax.ShapeDtypeStruct((N,), jnp.float32),
        mesh=mesh,
        scratch_types=[
            pltpu.VMEM_SHARED((SEQ_LEN,), jnp.float32),
            pltpu.VMEM((SEQ_LEN,), jnp.float32),
            pltpu.VMEM((CHUNK,), jnp.int32),
            pltpu.VMEM((CHUNK,), jnp.int32),
            pltpu.VMEM((CHUNK,), jnp.float32),
            pltpu.VMEM((CHUNK,), jnp.float32),
            pltpu.VMEM((CHUNK,), jnp.float32),
            pltpu.VMEM((CHUNK,), jnp.float32),
            pltpu.SemaphoreType.DMA((NBUF,)),
            pltpu.SemaphoreType.DMA((NBUF,)),
            pltpu.SemaphoreType.DMA,
        ],
        compiler_params=cp,
    )
    out = run(idx, src, rng_seq)
    return out.reshape(shape)
